# Initial kernel scaffold; baseline (speedup 1.0000x reference)
import jax
import jax.numpy as jnp
from jax.experimental import pallas as pl


def kernel(cls_tokens, feats, Wh, bh, Wt, bt, W1, b1, W2, b2, gamma, beta):
    raise NotImplementedError("write your pallas kernel here")


# speedup vs baseline: 3.2777x; 3.2777x over previous
import functools

import jax
import jax.numpy as jnp
import numpy as np
from jax.experimental import pallas as pl
from jax.experimental.pallas import tpu as pltpu

H_BLK = 16


def _mm(x, w):
    return jax.lax.dot_general(x, w, (((1,), (0,)), ((), ())),
                               preferred_element_type=jnp.float32)


def _mm_t(x, w):
    return jax.lax.dot_general(x, w, (((1,), (1,)), ((), ())),
                               preferred_element_type=jnp.float32)


def _lane_cumsum(x):
    r, n = x.shape
    s = 1
    while s < n:
        shifted = jnp.concatenate(
            [jnp.zeros((r, s), dtype=x.dtype), x[:, : n - s]], axis=1)
        x = x + shifted
        s *= 2
    return x


def _leaky(x):
    return jnp.where(x >= 0, x, 0.01 * x)


def _body(cls_ref, feats_ref, wh_ref, bh_ref, wt_ref, bt_ref, w1_ref, b1_ref,
          w2_ref, b2_ref, gamma_ref, beta_ref, out_ref,
          et_hi_ref, et_lo_ref, enh_ref, *, k):
    cls = cls_ref[0]
    feats = feats_ref[0]
    h_dim, d = cls.shape
    t_dim = feats.shape[0]
    n = h_dim + t_dim
    scale = d ** (-0.5)

    e_h = _mm_t(cls, wh_ref[...]) + bh_ref[...]
    e_t_cls = _mm_t(cls, wt_ref[...]) + bt_ref[...]
    e_t_feats = _mm_t(feats, wt_ref[...]) + bt_ref[...]

    for rows, base in ((e_t_cls, 0), (e_t_feats, h_dim)):
        hi = rows.astype(jnp.bfloat16)
        lo = (rows - hi.astype(jnp.float32)).astype(jnp.bfloat16)
        et_hi_ref[pl.ds(base, rows.shape[0]), :] = hi
        et_lo_ref[pl.ds(base, rows.shape[0]), :] = lo

    e_hs = e_h * scale
    attn = jnp.concatenate(
        [_mm_t(e_hs, e_t_cls), _mm_t(e_hs, e_t_feats)], axis=1)

    bits = jax.lax.bitcast_convert_type(attn, jnp.int32)
    skey = bits ^ jax.lax.shift_right_arithmetic(
        bits, 31).astype(jnp.int32) & jnp.int32(0x7FFFFFFF) ^ jnp.int32(0)
    skey = bits ^ (jax.lax.shift_right_arithmetic(bits, 31)
                   & jnp.int32(0x7FFFFFFF))
    sign = jnp.int32(np.int32(np.uint32(0x80000000)))
    prefix_u = jnp.zeros((h_dim, 1), dtype=jnp.int32)
    for b in range(31, -1, -1):
        bitc = jnp.int32(np.int32(np.uint32(1 << b)))
        cand_u = prefix_u | bitc
        cand_s = cand_u ^ sign
        cnt = jnp.sum((skey >= cand_s).astype(jnp.int32), axis=1,
                      keepdims=True)
        prefix_u = jnp.where(cnt >= k, cand_u, prefix_u)
    thr_s = prefix_u ^ sign

    mask_gt = skey > thr_s
    mask_eq = skey == thr_s
    cnt_gt = jnp.sum(mask_gt.astype(jnp.float32), axis=1, keepdims=True)
    need = jnp.float32(k) - cnt_gt
    rank_eq = _lane_cumsum(mask_eq.astype(jnp.float32))
    mask = mask_gt | (mask_eq & (rank_eq <= need))
    rank = _lane_cumsum(mask.astype(jnp.int32))
    rank_sel = jnp.where(mask, rank - 1, -1)

    n_blk = h_dim // H_BLK
    for blk in range(n_blk):
        h0 = blk * H_BLK
        rk = rank_sel[h0:h0 + H_BLK, :]
        rk3 = jnp.broadcast_to(rk[:, None, :], (H_BLK, k, n))
        r_iota = jax.lax.broadcasted_iota(jnp.int32, (H_BLK, k, n), 1)
        c_sel = (rk3 == r_iota).astype(jnp.bfloat16).reshape(H_BLK * k, n)
        nb = (_mm(c_sel, et_hi_ref[...]) + _mm(c_sel, et_lo_ref[...]))
        nb3 = nb.reshape(H_BLK, k, d)

        e_h_b = e_hs[h0:h0 + H_BLK, :][:, None, :]
        sel_logit = jnp.sum(nb3 * e_h_b, axis=2, keepdims=True)
        m = jnp.max(sel_logit, axis=1, keepdims=True)
        ex = jnp.exp(sel_logit - m)
        pk = ex / jnp.sum(ex, axis=1, keepdims=True)

        e_h_blk = e_h[h0:h0 + H_BLK, :][:, None, :]
        arg = (2.0 - pk) * e_h_blk + pk * nb3
        gate = jnp.tanh(arg)
        kaw = jnp.sum(nb3 * gate, axis=2, keepdims=True)
        km = jnp.max(kaw, axis=1, keepdims=True)
        kex = jnp.exp(kaw - km)
        ka_prob = kex / jnp.sum(kex, axis=1, keepdims=True)
        e_nh = jnp.sum(ka_prob * nb3, axis=1)
        enh_ref[pl.ds(h0, H_BLK), :] = e_nh

    e_nh_all = enh_ref[...]
    sum_in = (e_h + e_nh_all) * 0.1 + cls
    bi_in = e_h * e_nh_all * 0.1 + cls
    s_emb = _leaky(_mm_t(sum_in, w1_ref[...]) + b1_ref[...])
    b_emb = _leaky(_mm_t(bi_in, w2_ref[...]) + b2_ref[...])
    emb = s_emb + b_emb

    mu = jnp.mean(emb, axis=-1, keepdims=True)
    var = jnp.mean((emb - mu) ** 2, axis=-1, keepdims=True)
    out = (emb - mu) / jnp.sqrt(var + 1e-5) * gamma_ref[...] + beta_ref[...]
    out_ref[0] = out


@jax.jit
def kernel(cls_tokens, feats, Wh, bh, Wt, bt, W1, b1, W2, b2, gamma, beta):
    b_dim, h_dim, d = cls_tokens.shape
    t_dim = feats.shape[1]
    n = h_dim + t_dim
    k = max(1, min(t_dim, int(0.5 * max(1, h_dim))))

    row = lambda v: v.reshape(1, d)
    full = lambda shape: pl.BlockSpec(shape, lambda b: (0,) * len(shape))

    out = pl.pallas_call(
        functools.partial(_body, k=k),
        grid=(b_dim,),
        in_specs=[
            pl.BlockSpec((1, h_dim, d), lambda b: (b, 0, 0)),
            pl.BlockSpec((1, t_dim, d), lambda b: (b, 0, 0)),
            full((d, d)), full((1, d)),
            full((d, d)), full((1, d)),
            full((d, d)), full((1, d)),
            full((d, d)), full((1, d)),
            full((1, d)), full((1, d)),
        ],
        out_specs=pl.BlockSpec((1, h_dim, d), lambda b: (b, 0, 0)),
        out_shape=jax.ShapeDtypeStruct((b_dim, h_dim, d), jnp.float32),
        scratch_shapes=[
            pltpu.VMEM((n, d), jnp.bfloat16),
            pltpu.VMEM((n, d), jnp.bfloat16),
            pltpu.VMEM((h_dim, d), jnp.float32),
        ],
    )(cls_tokens, feats, Wh, row(bh), Wt, row(bt), W1, row(b1), W2, row(b2),
      row(gamma), row(beta))
    return out


# speedup vs baseline: 4.9676x; 1.5156x over previous
import functools

import jax
import jax.numpy as jnp
import numpy as np
from jax.experimental import pallas as pl
from jax.experimental.pallas import tpu as pltpu

H_BLK = 16


def _mm(x, w):
    return jax.lax.dot_general(x, w, (((1,), (0,)), ((), ())),
                               preferred_element_type=jnp.float32)


def _mm_t(x, w):
    return jax.lax.dot_general(x, w, (((1,), (1,)), ((), ())),
                               preferred_element_type=jnp.float32)


def _lane_cumsum(x):
    r, n = x.shape
    s = 1
    while s < n:
        shifted = jnp.concatenate(
            [jnp.zeros((r, s), dtype=x.dtype), x[:, : n - s]], axis=1)
        x = x + shifted
        s *= 2
    return x


def _leaky(x):
    return jnp.where(x >= 0, x, 0.01 * x)


def _body(cls_ref, feats_ref, wh_ref, bh_ref, wt_ref, bt_ref, w1_ref, b1_ref,
          w2_ref, b2_ref, gamma_ref, beta_ref, out_ref,
          et_ref, enh_ref, *, k):
    cls = cls_ref[0]
    feats = feats_ref[0]
    h_dim, d = cls.shape
    t_dim = feats.shape[0]
    n = h_dim + t_dim
    scale = d ** (-0.5)

    e_h = _mm_t(cls, wh_ref[...]) + bh_ref[...]
    e_t_cls = _mm_t(cls, wt_ref[...]) + bt_ref[...]
    e_t_feats = _mm_t(feats, wt_ref[...]) + bt_ref[...]

    et_ref[pl.ds(0, h_dim), :] = e_t_cls.astype(jnp.bfloat16)
    et_ref[pl.ds(h_dim, t_dim), :] = e_t_feats.astype(jnp.bfloat16)

    e_hs = e_h * scale
    attn = jnp.concatenate(
        [_mm_t(e_hs, e_t_cls), _mm_t(e_hs, e_t_feats)], axis=1)

    bits = jax.lax.bitcast_convert_type(attn, jnp.int32)
    skey = bits ^ jax.lax.shift_right_arithmetic(
        bits, 31).astype(jnp.int32) & jnp.int32(0x7FFFFFFF) ^ jnp.int32(0)
    skey = bits ^ (jax.lax.shift_right_arithmetic(bits, 31)
                   & jnp.int32(0x7FFFFFFF))
    sign = jnp.int32(np.int32(np.uint32(0x80000000)))
    prefix_u = jnp.zeros((h_dim, 1), dtype=jnp.int32)
    for b in range(31, -1, -1):
        bitc = jnp.int32(np.int32(np.uint32(1 << b)))
        cand_u = prefix_u | bitc
        cand_s = cand_u ^ sign
        cnt = jnp.sum((skey >= cand_s).astype(jnp.int32), axis=1,
                      keepdims=True)
        prefix_u = jnp.where(cnt >= k, cand_u, prefix_u)
    thr_s = prefix_u ^ sign

    mask_gt = skey > thr_s
    mask_eq = skey == thr_s
    cnt_gt = jnp.sum(mask_gt.astype(jnp.float32), axis=1, keepdims=True)
    need = jnp.float32(k) - cnt_gt
    rank_eq = _lane_cumsum(mask_eq.astype(jnp.float32))
    mask = mask_gt | (mask_eq & (rank_eq <= need))
    rank = _lane_cumsum(mask.astype(jnp.int32))
    rank_sel = jnp.where(mask, rank - 1, -1)

    n_blk = h_dim // H_BLK
    for blk in range(n_blk):
        h0 = blk * H_BLK
        rk = rank_sel[h0:h0 + H_BLK, :]
        rk3 = jnp.broadcast_to(rk[:, None, :], (H_BLK, k, n))
        r_iota = jax.lax.broadcasted_iota(jnp.int32, (H_BLK, k, n), 1)
        hit = rk3 == r_iota
        c_sel = hit.astype(jnp.bfloat16).reshape(H_BLK * k, n)
        nb = _mm(c_sel, et_ref[...])
        nb3 = nb.reshape(H_BLK, k, d)

        at_b = attn[h0:h0 + H_BLK, :][:, None, :]
        sel_logit = jnp.sum(jnp.where(hit, at_b, 0.0), axis=2, keepdims=True)
        m = jnp.max(sel_logit, axis=1, keepdims=True)
        ex = jnp.exp(sel_logit - m)
        pk = ex / jnp.sum(ex, axis=1, keepdims=True)

        e_h_blk = e_h[h0:h0 + H_BLK, :][:, None, :]
        arg = (2.0 - pk) * e_h_blk + pk * nb3
        gate = jnp.tanh(arg)
        kaw = jnp.sum(nb3 * gate, axis=2, keepdims=True)
        km = jnp.max(kaw, axis=1, keepdims=True)
        kex = jnp.exp(kaw - km)
        ka_prob = kex / jnp.sum(kex, axis=1, keepdims=True)
        e_nh = jnp.sum(ka_prob * nb3, axis=1)
        enh_ref[pl.ds(h0, H_BLK), :] = e_nh

    e_nh_all = enh_ref[...]
    sum_in = (e_h + e_nh_all) * 0.1 + cls
    bi_in = e_h * e_nh_all * 0.1 + cls
    s_emb = _leaky(_mm_t(sum_in, w1_ref[...]) + b1_ref[...])
    b_emb = _leaky(_mm_t(bi_in, w2_ref[...]) + b2_ref[...])
    emb = s_emb + b_emb

    mu = jnp.mean(emb, axis=-1, keepdims=True)
    var = jnp.mean((emb - mu) ** 2, axis=-1, keepdims=True)
    out = (emb - mu) / jnp.sqrt(var + 1e-5) * gamma_ref[...] + beta_ref[...]
    out_ref[0] = out


@jax.jit
def kernel(cls_tokens, feats, Wh, bh, Wt, bt, W1, b1, W2, b2, gamma, beta):
    b_dim, h_dim, d = cls_tokens.shape
    t_dim = feats.shape[1]
    n = h_dim + t_dim
    k = max(1, min(t_dim, int(0.5 * max(1, h_dim))))

    row = lambda v: v.reshape(1, d)
    full = lambda shape: pl.BlockSpec(shape, lambda b: (0,) * len(shape))

    out = pl.pallas_call(
        functools.partial(_body, k=k),
        grid=(b_dim,),
        in_specs=[
            pl.BlockSpec((1, h_dim, d), lambda b: (b, 0, 0)),
            pl.BlockSpec((1, t_dim, d), lambda b: (b, 0, 0)),
            full((d, d)), full((1, d)),
            full((d, d)), full((1, d)),
            full((d, d)), full((1, d)),
            full((d, d)), full((1, d)),
            full((1, d)), full((1, d)),
        ],
        out_specs=pl.BlockSpec((1, h_dim, d), lambda b: (b, 0, 0)),
        out_shape=jax.ShapeDtypeStruct((b_dim, h_dim, d), jnp.float32),
        scratch_shapes=[
            pltpu.VMEM((n, d), jnp.bfloat16),
            pltpu.VMEM((h_dim, d), jnp.float32),
        ],
    )(cls_tokens, feats, Wh, row(bh), Wt, row(bt), W1, row(b1), W2, row(b2),
      row(gamma), row(beta))
    return out
